# baseline (speedup 1.0000x reference)
import functools

import jax
import jax.numpy as jnp
from jax.experimental import pallas as pl
from jax.experimental.pallas import tpu as pltpu

NUM_TOKENS = 32768
HIDDEN = 768
NUM_EXPERTS = 8
LANES = 128
BLOCK = 2048
GRID = NUM_TOKENS // BLOCK


def _router_kernel(x_ref, wt_ref, logits_ref, sel_ref, wgt_ref, var_ref,
                   ent_ref, load_acc, ent_acc):
    i = pl.program_id(0)

    x = x_ref[...]
    wt = wt_ref[...]
    logits = jnp.dot(x, wt, preferred_element_type=jnp.float32)

    col = jax.lax.broadcasted_iota(jnp.int32, (BLOCK, LANES), 1)
    valid = col < NUM_EXPERTS
    masked = jnp.where(valid, logits, -1e30)

    m = jnp.max(masked, axis=1, keepdims=True)
    e = jnp.exp(masked - m)
    s = jnp.sum(e, axis=1, keepdims=True)
    probs = e / s

    logits_ref[...] = logits[:, :NUM_EXPERTS]
    sel_ref[...] = jnp.argmax(masked, axis=1).astype(jnp.int32)[:, None]
    wgt_ref[...] = 1.0 / s

    ent_tok = -jnp.sum(probs * jnp.log(probs + 1e-8), axis=1, keepdims=True)
    ent_part = jnp.sum(ent_tok).reshape(1, 1)
    load_part = jnp.sum(probs, axis=0, keepdims=True)

    @pl.when(i == 0)
    def _init():
        load_acc[...] = load_part
        ent_acc[...] = ent_part

    @pl.when(i > 0)
    def _accum():
        load_acc[...] += load_part
        ent_acc[...] += ent_part

    @pl.when(i == GRID - 1)
    def _finalize():
        load = load_acc[...] / NUM_TOKENS
        vmask = (jax.lax.broadcasted_iota(jnp.int32, (1, LANES), 1)
                 < NUM_EXPERTS).astype(jnp.float32)
        mean = jnp.sum(load * vmask) / NUM_EXPERTS
        var = jnp.sum(vmask * (load - mean) ** 2) / NUM_EXPERTS
        var_ref[...] = var.reshape(1, 1)
        ent_ref[...] = ent_acc[...] / NUM_TOKENS


@jax.jit
def kernel(hidden_states, W):
    wt = jnp.pad(W.T, ((0, 0), (0, LANES - NUM_EXPERTS)))

    out_types = (
        jax.ShapeDtypeStruct((NUM_TOKENS, NUM_EXPERTS), jnp.float32),
        jax.ShapeDtypeStruct((NUM_TOKENS, 1), jnp.int32),
        jax.ShapeDtypeStruct((NUM_TOKENS, 1), jnp.float32),
        jax.ShapeDtypeStruct((1, 1), jnp.float32),
        jax.ShapeDtypeStruct((1, 1), jnp.float32),
    )
    logits, sel, wgt, var, ent = pl.pallas_call(
        _router_kernel,
        grid=(GRID,),
        in_specs=[
            pl.BlockSpec((BLOCK, HIDDEN), lambda i: (i, 0)),
            pl.BlockSpec((HIDDEN, LANES), lambda i: (0, 0)),
        ],
        out_specs=(
            pl.BlockSpec((BLOCK, NUM_EXPERTS), lambda i: (i, 0)),
            pl.BlockSpec((BLOCK, 1), lambda i: (i, 0)),
            pl.BlockSpec((BLOCK, 1), lambda i: (i, 0)),
            pl.BlockSpec((1, 1), lambda i: (0, 0)),
            pl.BlockSpec((1, 1), lambda i: (0, 0)),
        ),
        out_shape=out_types,
        scratch_shapes=[
            pltpu.VMEM((1, LANES), jnp.float32),
            pltpu.VMEM((1, 1), jnp.float32),
        ],
    )(hidden_states, wt)

    return (logits, sel, wgt, var.reshape(()), ent.reshape(()))
